# Initial kernel scaffold; baseline (speedup 1.0000x reference)
import jax
import jax.numpy as jnp
from jax.experimental import pallas as pl


def kernel(queries, candidates):
    raise NotImplementedError("write your pallas kernel here")


# speedup vs baseline: 1.0463x; 1.0463x over previous
import functools

import jax
import jax.numpy as jnp
from jax.experimental import pallas as pl

K_TOP = 100


def _score_body(q_ref, c_ref, o_ref):
    o_ref[...] = jax.lax.dot_general(
        q_ref[...], c_ref[...],
        (((1,), (1,)), ((), ())),
        preferred_element_type=jnp.float32,
    )


def kernel(queries, candidates):
    Q, D = queries.shape
    N = candidates.shape[0]
    QB = 512
    CB = 2048
    grid = (Q // QB, pl.cdiv(N, CB))
    scores = pl.pallas_call(
        _score_body,
        grid=grid,
        in_specs=[
            pl.BlockSpec((QB, D), lambda i, j: (i, 0)),
            pl.BlockSpec((CB, D), lambda i, j: (j, 0)),
        ],
        out_specs=pl.BlockSpec((QB, CB), lambda i, j: (i, j)),
        out_shape=jax.ShapeDtypeStruct((Q, N), jnp.float32),
    )(queries, candidates)
    s, i = jax.lax.top_k(scores, K_TOP)
    return s, i.astype(jnp.int32)


# speedup vs baseline: 10.3871x; 9.9279x over previous
import functools

import jax
import jax.numpy as jnp
from jax.experimental import pallas as pl

K_TOP = 100
QB = 512
CB = 2048
GS_SUB = 16
E_EXTRACT = 4
SLOTS = 16
NEG_INF = float("-inf")
MIN_I32 = -2147483648


def _score_block(q_ref, c_ref):
    return jax.lax.dot_general(
        q_ref[...], c_ref[...],
        (((1,), (1,)), ((), ())),
        preferred_element_type=jnp.float32,
    )


def _masked_scores3(q_ref, c_ref, j, n_valid):
    s = _score_block(q_ref, c_ref)
    s3 = s.reshape(QB, GS_SUB, 128)
    sub_iota = jax.lax.broadcasted_iota(jnp.int32, (QB, GS_SUB, 128), 1)
    lane_iota3 = jax.lax.broadcasted_iota(jnp.int32, (QB, GS_SUB, 128), 2)
    gid3 = j * CB + sub_iota * 128 + lane_iota3
    return jnp.where(gid3 < n_valid, s3, NEG_INF), sub_iota


def _groupmax_body(q_ref, c_ref, o_ref, *, n_valid):
    j = pl.program_id(1)
    s3, _ = _masked_scores3(q_ref, c_ref, j, n_valid)
    o_ref[...] = jnp.max(s3, axis=1)


def _float_to_sortable_i32(x):
    b = jax.lax.bitcast_convert_type(x, jnp.int32)
    return jnp.where(b >= 0, b, b ^ jnp.int32(0x7FFFFFFF))


def _sortable_i32_to_float(s):
    b = jnp.where(s >= 0, s, s ^ jnp.int32(0x7FFFFFFF))
    return jax.lax.bitcast_convert_type(b, jnp.float32)


def _kth_body(gm_ref, o_ref):
    s = _float_to_sortable_i32(gm_ref[...])

    def round_fn(i, p):
        b = 31 - i
        test = p | (jnp.int32(1) << b)
        tst_s = test ^ MIN_I32
        cnt = jnp.sum((s >= tst_s).astype(jnp.int32), axis=1, keepdims=True)
        return jnp.where(cnt >= K_TOP, test, p)

    p = jax.lax.fori_loop(0, 32, round_fn, jnp.zeros((QB, 1), jnp.int32))
    o_ref[...] = jnp.broadcast_to(p ^ MIN_I32, (QB, 128))


def _extract_body(q_ref, c_ref, t_ref, v_ref, i_ref, n_ref, *, n_valid):
    j = pl.program_id(1)
    t = t_ref[...][:, :1]
    s3, sub_iota = _masked_scores3(q_ref, c_ref, j, n_valid)

    ge = (s3 >= t[:, :, None]).astype(jnp.int32)
    cnt_g = jnp.sum(ge, axis=1)
    blk_cnt = jnp.sum(cnt_g, axis=1, keepdims=True)
    max_g = jnp.max(cnt_g, axis=1, keepdims=True)

    lane_iota = jax.lax.broadcasted_iota(jnp.int32, (QB, 128), 1)

    vals = []
    ids = []
    work = s3
    for _ in range(E_EXTRACT):
        m = jnp.max(work, axis=1)
        eq = work == m[:, None, :]
        am = jnp.min(jnp.where(eq, sub_iota, GS_SUB), axis=1)
        gid = j * CB + am * 128 + lane_iota
        keep = m >= t
        vals.append(jnp.where(keep, m, NEG_INF))
        ids.append(jnp.where(keep, gid, 0))
        work = jnp.where(sub_iota == am[:, None, :], NEG_INF, work)

    V = jnp.concatenate(vals, axis=1)
    I = jnp.concatenate(ids, axis=1)
    stack_iota = jax.lax.broadcasted_iota(jnp.int32, (QB, E_EXTRACT * 128), 1)

    out_v = []
    out_i = []
    for _ in range(SLOTS):
        mx = jnp.max(V, axis=1, keepdims=True)
        eq = V == mx
        pos = jnp.min(jnp.where(eq, stack_iota, E_EXTRACT * 128),
                      axis=1, keepdims=True)
        hit = stack_iota == pos
        out_v.append(mx)
        out_i.append(jnp.sum(jnp.where(hit, I, 0), axis=1, keepdims=True))
        V = jnp.where(hit, NEG_INF, V)

    v_ref[...] = jnp.concatenate(out_v, axis=1)[None]
    i_ref[...] = jnp.concatenate(out_i, axis=1)[None]
    n_ref[...] = jnp.concatenate(
        [blk_cnt, max_g] + [jnp.zeros((QB, 1), jnp.int32)] * (SLOTS - 2),
        axis=1)[None]


def _full_score_body(q_ref, c_ref, o_ref):
    o_ref[...] = _score_block(q_ref, c_ref)


def _full_topk(queries, cand_padded, n_valid):
    Q, D = queries.shape
    Np = cand_padded.shape[0]
    scores = pl.pallas_call(
        _full_score_body,
        grid=(Q // QB, Np // CB),
        in_specs=[
            pl.BlockSpec((QB, D), lambda i, j: (i, 0)),
            pl.BlockSpec((CB, D), lambda i, j: (j, 0)),
        ],
        out_specs=pl.BlockSpec((QB, CB), lambda i, j: (i, j)),
        out_shape=jax.ShapeDtypeStruct((Q, Np), jnp.float32),
    )(queries, cand_padded)
    s, i = jax.lax.top_k(scores[:, :n_valid], K_TOP)
    return s, i.astype(jnp.int32)


def kernel(queries, candidates):
    Q, D = queries.shape
    N = candidates.shape[0]
    n_pad = (-N) % CB
    if n_pad:
        candidates = jnp.pad(candidates, ((0, n_pad), (0, 0)))
    Np = N + n_pad
    nblk = Np // CB
    G = Np // GS_SUB

    gmax = pl.pallas_call(
        functools.partial(_groupmax_body, n_valid=N),
        grid=(Q // QB, nblk),
        in_specs=[
            pl.BlockSpec((QB, D), lambda i, j: (i, 0)),
            pl.BlockSpec((CB, D), lambda i, j: (j, 0)),
        ],
        out_specs=pl.BlockSpec((QB, 128), lambda i, j: (i, j)),
        out_shape=jax.ShapeDtypeStruct((Q, G), jnp.float32),
    )(queries, candidates)

    kth_s = pl.pallas_call(
        _kth_body,
        grid=(Q // QB,),
        in_specs=[pl.BlockSpec((QB, G), lambda i: (i, 0))],
        out_specs=pl.BlockSpec((QB, 128), lambda i: (i, 0)),
        out_shape=jax.ShapeDtypeStruct((Q, 128), jnp.int32),
    )(gmax)

    thresh = _sortable_i32_to_float(kth_s[:, :1])
    thresh128 = jnp.broadcast_to(thresh, (Q, 128))

    vals, ids, cnts = pl.pallas_call(
        functools.partial(_extract_body, n_valid=N),
        grid=(Q // QB, nblk),
        in_specs=[
            pl.BlockSpec((QB, D), lambda i, j: (i, 0)),
            pl.BlockSpec((CB, D), lambda i, j: (j, 0)),
            pl.BlockSpec((QB, 128), lambda i, j: (i, 0)),
        ],
        out_specs=[
            pl.BlockSpec((1, QB, SLOTS), lambda i, j: (j, i, 0)),
            pl.BlockSpec((1, QB, SLOTS), lambda i, j: (j, i, 0)),
            pl.BlockSpec((1, QB, SLOTS), lambda i, j: (j, i, 0)),
        ],
        out_shape=[
            jax.ShapeDtypeStruct((nblk, Q, SLOTS), jnp.float32),
            jax.ShapeDtypeStruct((nblk, Q, SLOTS), jnp.int32),
            jax.ShapeDtypeStruct((nblk, Q, SLOTS), jnp.int32),
        ],
    )(queries, candidates, thresh128)

    vals = jnp.transpose(vals, (1, 0, 2)).reshape(Q, nblk * SLOTS)
    ids = jnp.transpose(ids, (1, 0, 2)).reshape(Q, nblk * SLOTS)
    overflow = jnp.logical_or(
        jnp.max(cnts[:, :, 0]) > SLOTS,
        jnp.max(cnts[:, :, 1]) > E_EXTRACT,
    )

    def small_path(_):
        s, p = jax.lax.top_k(vals, K_TOP)
        return s, jnp.take_along_axis(ids, p, axis=1)

    def fallback_path(_):
        return _full_topk(queries, candidates, N)

    return jax.lax.cond(overflow, fallback_path, small_path, operand=None)
